# baseline (speedup 1.0000x reference)
import jax
import jax.numpy as jnp
from jax.experimental import pallas as pl
from jax.experimental.pallas import tpu as pltpu

N0, N1 = 2048, 4096
D = 128
BI = 1024
BJ = 1024
GI = N0 // BI
GJ = N1 // BJ
A0K = N0 // GJ
A1K = N1 // GI

_F32 = jnp.float32


def _gnn_kernel(x0_ref, x1_ref, a0_ref, a1_ref, b_ref,
                w0_ref, w1_ref, ws_ref, wt_ref,
                out0_ref, out1_ref,
                y0_ref, y1_ref, ys_ref, ytT_ref, accT_ref):
    j = pl.program_id(0)
    i = pl.program_id(1)

    _bf = jnp.bfloat16

    @pl.when(i == 0)
    def _proj_j():
        y0_ref[pl.ds(j * A0K, A0K), :] = jnp.dot(
            x0_ref[pl.ds(j * A0K, A0K), :], w0_ref[...],
            preferred_element_type=_F32).astype(_bf)
        ys_ref[pl.ds(j * BJ, BJ), :] = jnp.dot(
            x1_ref[pl.ds(j * BJ, BJ), :], ws_ref[...],
            preferred_element_type=_F32).astype(_bf)

    @pl.when(j == 0)
    def _proj_i():
        y1_ref[pl.ds(i * A1K, A1K), :] = jnp.dot(
            x1_ref[pl.ds(i * A1K, A1K), :], w1_ref[...],
            preferred_element_type=_F32).astype(_bf)
        yt = jnp.dot(x0_ref[pl.ds(i * BI, BI), :], wt_ref[...],
                     preferred_element_type=_F32).astype(_bf)
        ytT_ref[:, pl.ds(i * BI, BI)] = yt.T

    b = b_ref[...].astype(_bf)

    m0 = jnp.dot(a0_ref[...].astype(_bf),
                 y0_ref[pl.ds(j * A0K, A0K), :],
                 preferred_element_type=_F32)
    m0 += jnp.dot(b, ys_ref[pl.ds(j * BJ, BJ), :],
                  preferred_element_type=_F32)

    m1 = jnp.dot(a1_ref[...].astype(_bf),
                 y1_ref[pl.ds(i * A1K, A1K), :],
                 preferred_element_type=_F32)
    m1T = jnp.dot(ytT_ref[:, pl.ds(i * BI, BI)], b,
                  preferred_element_type=_F32)

    @pl.when(j == 0)
    def _():
        out0_ref[pl.ds(i * BI, BI), :] = m0

    @pl.when(j > 0)
    def _():
        out0_ref[pl.ds(i * BI, BI), :] += m0

    if GI == 1:
        out1_ref[...] = m1 + m1T.T
    else:
        @pl.when(i == 0)
        def _():
            accT_ref[...] = m1T
            out1_ref[...] = m1

        if GI > 2:
            @pl.when(jnp.logical_and(i > 0, i < GI - 1))
            def _():
                accT_ref[...] += m1T
                out1_ref[...] += m1

        @pl.when(i == GI - 1)
        def _():
            totT = accT_ref[...] + m1T
            out1_ref[...] += m1 + totT.T


def kernel(x_0, x_1, x_2, x_3, x_4, adjacency_0, adjacency_1, adjacency_2,
           adjacency_3, adjacency_4, incidence_0_1, incidence_0_2,
           incidence_0_3, incidence_0_4, incidence_1_2, incidence_1_3,
           incidence_1_4, incidence_2_3, incidence_2_4, incidence_3_4,
           w_hbs0, w_hbs1, w_hbns_s, w_hbns_t):
    grid = (GJ, GI)
    full = lambda shape: pl.BlockSpec(shape, lambda j, i: (0, 0))
    out0, out1 = pl.pallas_call(
        _gnn_kernel,
        grid=grid,
        in_specs=[
            full((N0, D)),
            full((N1, D)),
            pl.BlockSpec((BI, A0K), lambda j, i: (i, j)),
            pl.BlockSpec((BJ, A1K), lambda j, i: (j, i)),
            pl.BlockSpec((BI, BJ), lambda j, i: (i, j)),
            full((D, D)), full((D, D)), full((D, D)), full((D, D)),
        ],
        out_specs=[
            full((N0, D)),
            pl.BlockSpec((BJ, D), lambda j, i: (j, 0)),
        ],
        out_shape=[
            jax.ShapeDtypeStruct((N0, D), _F32),
            jax.ShapeDtypeStruct((N1, D), _F32),
        ],
        scratch_shapes=[
            pltpu.VMEM((N0, D), jnp.bfloat16),
            pltpu.VMEM((N1, D), jnp.bfloat16),
            pltpu.VMEM((N1, D), jnp.bfloat16),
            pltpu.VMEM((D, N0), jnp.bfloat16),
            pltpu.VMEM((D, BJ), _F32),
        ],
    )(x_0, x_1, adjacency_0, adjacency_1, incidence_0_1,
      w_hbs0, w_hbs1, w_hbns_s, w_hbns_t)
    return (out0, out1, x_2, x_3, x_4)


# speedup vs baseline: 1.0051x; 1.0051x over previous
import jax
import jax.numpy as jnp
from jax.experimental import pallas as pl
from jax.experimental.pallas import tpu as pltpu

N0, N1 = 2048, 4096
D = 128
BI = 1024
BJ = 1024
GI = N0 // BI
GJ = N1 // BJ
A0K = N0 // GJ
A1K = N1 // GI

_F32 = jnp.float32


def _gnn_kernel(x0_ref, x1_ref, a0_ref, a1_ref, b_ref,
                w0_ref, w1_ref, ws_ref, wt_ref,
                out0_ref, out1_ref,
                y0_ref, y1_ref, ys_ref, ytT_ref, accT_ref):
    j = pl.program_id(0)
    i = pl.program_id(1)

    @pl.when(i == 0)
    def _proj_j():
        y0_ref[pl.ds(j * A0K, A0K), :] = jnp.dot(
            x0_ref[pl.ds(j * A0K, A0K), :], w0_ref[...],
            preferred_element_type=_F32)
        ys_ref[pl.ds(j * BJ, BJ), :] = jnp.dot(
            x1_ref[pl.ds(j * BJ, BJ), :], ws_ref[...],
            preferred_element_type=_F32)

    @pl.when(j == 0)
    def _proj_i():
        y1_ref[pl.ds(i * A1K, A1K), :] = jnp.dot(
            x1_ref[pl.ds(i * A1K, A1K), :], w1_ref[...],
            preferred_element_type=_F32)
        yt = jnp.dot(x0_ref[pl.ds(i * BI, BI), :], wt_ref[...],
                     preferred_element_type=_F32)
        ytT_ref[:, pl.ds(i * BI, BI)] = yt.T

    _bf = jnp.bfloat16
    b = b_ref[...].astype(_bf)

    m0 = jnp.dot(a0_ref[...].astype(_bf),
                 y0_ref[pl.ds(j * A0K, A0K), :].astype(_bf),
                 preferred_element_type=_F32)
    m0 += jnp.dot(b, ys_ref[pl.ds(j * BJ, BJ), :].astype(_bf),
                  preferred_element_type=_F32)

    m1 = jnp.dot(a1_ref[...].astype(_bf),
                 y1_ref[pl.ds(i * A1K, A1K), :].astype(_bf),
                 preferred_element_type=_F32)
    m1T = jnp.dot(ytT_ref[:, pl.ds(i * BI, BI)].astype(_bf), b,
                  preferred_element_type=_F32)

    @pl.when(j == 0)
    def _():
        out0_ref[pl.ds(i * BI, BI), :] = m0

    @pl.when(j > 0)
    def _():
        out0_ref[pl.ds(i * BI, BI), :] += m0

    if GI == 1:
        out1_ref[...] = m1 + m1T.T
    else:
        @pl.when(i == 0)
        def _():
            accT_ref[...] = m1T
            out1_ref[...] = m1

        if GI > 2:
            @pl.when(jnp.logical_and(i > 0, i < GI - 1))
            def _():
                accT_ref[...] += m1T
                out1_ref[...] += m1

        @pl.when(i == GI - 1)
        def _():
            totT = accT_ref[...] + m1T
            out1_ref[...] += m1 + totT.T


def kernel(x_0, x_1, x_2, x_3, x_4, adjacency_0, adjacency_1, adjacency_2,
           adjacency_3, adjacency_4, incidence_0_1, incidence_0_2,
           incidence_0_3, incidence_0_4, incidence_1_2, incidence_1_3,
           incidence_1_4, incidence_2_3, incidence_2_4, incidence_3_4,
           w_hbs0, w_hbs1, w_hbns_s, w_hbns_t):
    grid = (GJ, GI)
    full = lambda shape: pl.BlockSpec(shape, lambda j, i: (0, 0))
    out0, out1 = pl.pallas_call(
        _gnn_kernel,
        grid=grid,
        in_specs=[
            full((N0, D)),
            full((N1, D)),
            pl.BlockSpec((BI, A0K), lambda j, i: (i, j)),
            pl.BlockSpec((BJ, A1K), lambda j, i: (j, i)),
            pl.BlockSpec((BI, BJ), lambda j, i: (i, j)),
            full((D, D)), full((D, D)), full((D, D)), full((D, D)),
        ],
        out_specs=[
            full((N0, D)),
            pl.BlockSpec((BJ, D), lambda j, i: (j, 0)),
        ],
        out_shape=[
            jax.ShapeDtypeStruct((N0, D), _F32),
            jax.ShapeDtypeStruct((N1, D), _F32),
        ],
        scratch_shapes=[
            pltpu.VMEM((N0, D), _F32),
            pltpu.VMEM((N1, D), _F32),
            pltpu.VMEM((N1, D), _F32),
            pltpu.VMEM((D, N0), _F32),
            pltpu.VMEM((D, BJ), _F32),
        ],
    )(x_0, x_1, adjacency_0, adjacency_1, incidence_0_1,
      w_hbs0, w_hbs1, w_hbns_s, w_hbns_t)
    return (out0, out1, x_2, x_3, x_4)
